# baseline (speedup 1.0000x reference)
import functools

import jax
import jax.numpy as jnp
from jax import lax
from jax.experimental import pallas as pl
from jax.experimental.pallas import tpu as pltpu
from jax.experimental.pallas import tpu_sc as plsc

NUM_CORES = 2
NUM_SUBCORES = 16
LANES = 16
CHUNK = 128


def _make_sc_kernel(B, D):
    NW = NUM_CORES * NUM_SUBCORES
    b_per_w = B // NW
    n_ch = b_per_w // CHUNK
    n_blk = b_per_w // LANES

    mesh = plsc.VectorSubcoreMesh(
        core_axis_name="c", subcore_axis_name="s",
        num_cores=NUM_CORES, num_subcores=NUM_SUBCORES)

    @functools.partial(
        pl.kernel,
        mesh=mesh,
        out_type=(
            jax.ShapeDtypeStruct((B, D), jnp.float32),
            jax.ShapeDtypeStruct((B, D), jnp.float32),
            jax.ShapeDtypeStruct((B,), jnp.float32),
        ),
        scratch_types=[
            pltpu.VMEM((n_ch, CHUNK), jnp.int32),
            pltpu.VMEM((n_ch, CHUNK), jnp.int32),
            pltpu.VMEM((b_per_w, D), jnp.float32),
            pltpu.VMEM((b_per_w, D), jnp.float32),
            pltpu.VMEM((b_per_w,), jnp.float32),
            pltpu.SemaphoreType.DMA,
            pltpu.SemaphoreType.DMA,
            pltpu.SemaphoreType.DMA,
            pltpu.SemaphoreType.DMA,
        ],
        compiler_params=pltpu.CompilerParams(
            needs_layout_passes=False, use_tc_tiling_on_sc=False),
    )
    def sc_kernel(user_hbm, item_hbm, ut_hbm, it_hbm,
                  eu_out, ei_out, rec_out,
                  idx_u, idx_i, eu_v, ei_v, rec_v,
                  sem_idx, sem_u, sem_i, sem_out):
        wid = lax.axis_index("s") * NUM_CORES + lax.axis_index("c")
        base = wid * b_per_w

        idx_copies = []
        for c in range(n_ch):
            idx_copies.append(pltpu.async_copy(
                user_hbm.at[pl.ds(base + c * CHUNK, CHUNK)],
                idx_u.at[c], sem_idx))
            idx_copies.append(pltpu.async_copy(
                item_hbm.at[pl.ds(base + c * CHUNK, CHUNK)],
                idx_i.at[c], sem_idx))
        for cp in idx_copies:
            cp.wait()

        gathers = []
        for c in range(n_ch):
            gathers.append(pltpu.async_copy(
                ut_hbm.at[idx_u.at[c]],
                eu_v.at[pl.ds(c * CHUNK, CHUNK)], sem_u))
            gathers.append(pltpu.async_copy(
                it_hbm.at[idx_i.at[c]],
                ei_v.at[pl.ds(c * CHUNK, CHUNK)], sem_i))
        for cp in gathers:
            cp.wait()

        out_copies = [
            pltpu.async_copy(eu_v, eu_out.at[pl.ds(base, b_per_w)], sem_out),
            pltpu.async_copy(ei_v, ei_out.at[pl.ds(base, b_per_w)], sem_out),
        ]

        lane = lax.iota(jnp.int32, LANES)

        def blk_body(b, _):
            row = b * LANES + lane
            acc = jnp.zeros((LANES,), jnp.float32)
            for d in range(D):
                col = jnp.full((LANES,), d, jnp.int32)
                u = plsc.load_gather(eu_v, [row, col])
                v = plsc.load_gather(ei_v, [row, col])
                acc = acc + u * v
            rec_v[pl.ds(b * LANES, LANES)] = acc
            return 0

        lax.fori_loop(0, n_blk, blk_body, 0)

        pltpu.sync_copy(rec_v, rec_out.at[pl.ds(base, b_per_w)])
        for cp in out_copies:
            cp.wait()

    return sc_kernel


def kernel(user, item, user_table, item_table):
    B = user.shape[0]
    D = user_table.shape[1]
    sc = _make_sc_kernel(B, D)
    eu, ei, rec = sc(user.astype(jnp.int32), item.astype(jnp.int32),
                     user_table, item_table)
    return (eu, ei, rec)


# speedup vs baseline: 3.8852x; 3.8852x over previous
import functools

import jax
import jax.numpy as jnp
from jax import lax
from jax.experimental import pallas as pl
from jax.experimental.pallas import tpu as pltpu
from jax.experimental.pallas import tpu_sc as plsc

NUM_CORES = 2
NUM_SUBCORES = 16
LANES = 16
KG = 4
TILE_W = 128


def _make_sc_kernel(B, D, N):
    NW = NUM_CORES * NUM_SUBCORES
    b_per_w = B // NW
    n_grp = b_per_w // KG
    n_blk = b_per_w // LANES

    mesh = plsc.VectorSubcoreMesh(
        core_axis_name="c", subcore_axis_name="s",
        num_cores=NUM_CORES, num_subcores=NUM_SUBCORES)

    @functools.partial(
        pl.kernel,
        mesh=mesh,
        out_type=(
            jax.ShapeDtypeStruct((D * B,), jnp.float32),
            jax.ShapeDtypeStruct((D * B,), jnp.float32),
            jax.ShapeDtypeStruct((B,), jnp.float32),
        ),
        scratch_types=[
            pltpu.SMEM((b_per_w,), jnp.int32),
            pltpu.SMEM((b_per_w,), jnp.int32),
            pltpu.VMEM((b_per_w,), jnp.int32),
            pltpu.VMEM((b_per_w,), jnp.int32),
            pltpu.VMEM((2, KG, D, TILE_W), jnp.float32),
            pltpu.VMEM((2, KG, D, TILE_W), jnp.float32),
            pltpu.VMEM((D * b_per_w,), jnp.float32),
            pltpu.VMEM((D * b_per_w,), jnp.float32),
            pltpu.VMEM((b_per_w,), jnp.float32),
            pltpu.SemaphoreType.DMA,
            pltpu.SemaphoreType.DMA,
            pltpu.SemaphoreType.DMA,
        ],
        compiler_params=pltpu.CompilerParams(
            needs_layout_passes=False, use_tc_tiling_on_sc=True),
    )
    def sc_kernel(user_hbm, item_hbm, ut_t_hbm, it_t_hbm,
                  eu_out, ei_out, rec_out,
                  ids_u, ids_i, idsv_u, idsv_i, ring_u, ring_i,
                  eu_v, ei_v, rec_v,
                  sem_idx, sem_u, sem_i):
        wid = lax.axis_index("s") * NUM_CORES + lax.axis_index("c")
        base = wid * b_per_w

        cp_u = pltpu.async_copy(
            user_hbm.at[pl.ds(base, b_per_w)], idsv_u, sem_idx)
        cp_i = pltpu.async_copy(
            item_hbm.at[pl.ds(base, b_per_w)], idsv_i, sem_idx)
        cp_u.wait()
        cp_i.wait()
        def smem_body(grp, _):
            o = grp * LANES
            vu = idsv_u[pl.ds(o, LANES)]
            vi = idsv_i[pl.ds(o, LANES)]
            for k in range(LANES):
                ids_u[o + k] = vu[k]
                ids_i[o + k] = vi[k]
            return 0

        lax.fori_loop(0, b_per_w // LANES, smem_body, 0)

        dlane = lax.iota(jnp.int32, LANES)
        dlo = dlane * b_per_w
        dhi = (dlane + LANES) * b_per_w

        def fire_group(g, half):
            for k in range(KG):
                j = g * KG + k
                tu = pl.multiple_of((ids_u[j] >> 7) * TILE_W, TILE_W)
                pltpu.async_copy(
                    ut_t_hbm.at[:, pl.ds(tu, TILE_W)],
                    ring_u.at[half, k], sem_u)
                ti = pl.multiple_of((ids_i[j] >> 7) * TILE_W, TILE_W)
                pltpu.async_copy(
                    it_t_hbm.at[:, pl.ds(ti, TILE_W)],
                    ring_i.at[half, k], sem_i)

        def drain_group(half):
            for k in range(KG):
                pltpu.make_async_copy(
                    ut_t_hbm.at[:, pl.ds(0, TILE_W)],
                    ring_u.at[half, k], sem_u).wait()
                pltpu.make_async_copy(
                    it_t_hbm.at[:, pl.ds(0, TILE_W)],
                    ring_i.at[half, k], sem_i).wait()

        def extract_group(g, half):
            hv = jnp.full((LANES,), half, jnp.int32)
            for k in range(KG):
                j = g * KG + k
                kv = jnp.full((LANES,), k, jnp.int32)
                jv = jnp.full((LANES,), j, jnp.int32)
                cu = jnp.full((LANES,), ids_u[j] & (TILE_W - 1), jnp.int32)
                u0 = plsc.load_gather(ring_u, [hv, kv, dlane, cu])
                u1 = plsc.load_gather(ring_u, [hv, kv, dlane + LANES, cu])
                plsc.store_scatter(eu_v, [dlo + jv], u0)
                plsc.store_scatter(eu_v, [dhi + jv], u1)
                ci = jnp.full((LANES,), ids_i[j] & (TILE_W - 1), jnp.int32)
                i0 = plsc.load_gather(ring_i, [hv, kv, dlane, ci])
                i1 = plsc.load_gather(ring_i, [hv, kv, dlane + LANES, ci])
                plsc.store_scatter(ei_v, [dlo + jv], i0)
                plsc.store_scatter(ei_v, [dhi + jv], i1)

        fire_group(0, 0)

        def grp_body(g, _):
            half = lax.rem(g, 2)
            nxt = lax.rem(g + 1, 2)

            @pl.when(g + 1 < n_grp)
            def _():
                fire_group(g + 1, nxt)

            drain_group(half)
            extract_group(g, half)
            return 0

        lax.fori_loop(0, n_grp, grp_body, 0)

        def blk_body(b, _):
            acc = jnp.zeros((LANES,), jnp.float32)
            for d in range(D):
                o = d * b_per_w + b * LANES
                acc = acc + (eu_v[pl.ds(o, LANES)] * ei_v[pl.ds(o, LANES)])
            rec_v[pl.ds(b * LANES, LANES)] = acc
            return 0

        lax.fori_loop(0, n_blk, blk_body, 0)

        out_copies = []
        for d in range(D):
            out_copies.append(pltpu.async_copy(
                eu_v.at[pl.ds(d * b_per_w, b_per_w)],
                eu_out.at[pl.ds(d * B + base, b_per_w)], sem_idx))
            out_copies.append(pltpu.async_copy(
                ei_v.at[pl.ds(d * b_per_w, b_per_w)],
                ei_out.at[pl.ds(d * B + base, b_per_w)], sem_idx))
        pltpu.sync_copy(rec_v, rec_out.at[pl.ds(base, b_per_w)])
        for cp in out_copies:
            cp.wait()

    return sc_kernel


def kernel(user, item, user_table, item_table):
    B = user.shape[0]
    N, D = user_table.shape
    sc = _make_sc_kernel(B, D, N)
    eu_f, ei_f, rec = sc(user.astype(jnp.int32), item.astype(jnp.int32),
                         user_table.T, item_table.T)
    return (eu_f.reshape(D, B).T, ei_f.reshape(D, B).T, rec)


# speedup vs baseline: 4.4506x; 1.1455x over previous
import functools

import jax
import jax.numpy as jnp
from jax import lax
from jax.experimental import pallas as pl
from jax.experimental.pallas import tpu as pltpu
from jax.experimental.pallas import tpu_sc as plsc

NUM_CORES = 2
NUM_SUBCORES = 16
LANES = 16
NSLOT = 10
TILE_W = 128


def _make_sc_kernel(B, D, N):
    NW = NUM_CORES * NUM_SUBCORES
    b_per_w = B // NW
    n_blk = b_per_w // LANES

    mesh = plsc.VectorSubcoreMesh(
        core_axis_name="c", subcore_axis_name="s",
        num_cores=NUM_CORES, num_subcores=NUM_SUBCORES)

    @functools.partial(
        pl.kernel,
        mesh=mesh,
        out_type=(
            jax.ShapeDtypeStruct((D * B,), jnp.float32),
            jax.ShapeDtypeStruct((D * B,), jnp.float32),
            jax.ShapeDtypeStruct((B,), jnp.float32),
        ),
        scratch_types=[
            pltpu.SMEM((b_per_w,), jnp.int32),
            pltpu.SMEM((b_per_w,), jnp.int32),
            pltpu.VMEM((b_per_w,), jnp.int32),
            pltpu.VMEM((b_per_w,), jnp.int32),
            pltpu.VMEM((NSLOT, D, TILE_W), jnp.float32),
            pltpu.VMEM((NSLOT, D, TILE_W), jnp.float32),
            pltpu.VMEM((D * b_per_w,), jnp.float32),
            pltpu.VMEM((D * b_per_w,), jnp.float32),
            pltpu.VMEM((b_per_w,), jnp.float32),
            pltpu.SemaphoreType.DMA,
            pltpu.SemaphoreType.DMA,
            pltpu.SemaphoreType.DMA,
        ],
        compiler_params=pltpu.CompilerParams(
            needs_layout_passes=False, use_tc_tiling_on_sc=True),
    )
    def sc_kernel(user_hbm, item_hbm, ut_t_hbm, it_t_hbm,
                  eu_out, ei_out, rec_out,
                  ids_u, ids_i, idsv_u, idsv_i, ring_u, ring_i,
                  eu_v, ei_v, rec_v,
                  sem_idx, sem_u, sem_i):
        wid = lax.axis_index("s") * NUM_CORES + lax.axis_index("c")
        base = wid * b_per_w

        cp_u = pltpu.async_copy(
            user_hbm.at[pl.ds(base, b_per_w)], idsv_u, sem_idx)
        cp_i = pltpu.async_copy(
            item_hbm.at[pl.ds(base, b_per_w)], idsv_i, sem_idx)
        cp_u.wait()
        cp_i.wait()
        def smem_body(grp, _):
            o = grp * LANES
            vu = idsv_u[pl.ds(o, LANES)]
            vi = idsv_i[pl.ds(o, LANES)]
            for k in range(LANES):
                ids_u[o + k] = vu[k]
                ids_i[o + k] = vi[k]
            return 0

        lax.fori_loop(0, b_per_w // LANES, smem_body, 0)

        dlane = lax.iota(jnp.int32, LANES)
        dlo = dlane * b_per_w
        dhi = (dlane + LANES) * b_per_w

        def fire(j, slot):
            tu = pl.multiple_of((ids_u[j] >> 7) * TILE_W, TILE_W)
            pltpu.async_copy(
                ut_t_hbm.at[:, pl.ds(tu, TILE_W)], ring_u.at[slot], sem_u)
            ti = pl.multiple_of((ids_i[j] >> 7) * TILE_W, TILE_W)
            pltpu.async_copy(
                it_t_hbm.at[:, pl.ds(ti, TILE_W)], ring_i.at[slot], sem_i)

        def drain_one():
            pltpu.make_async_copy(
                ut_t_hbm.at[:, pl.ds(0, TILE_W)], ring_u.at[0], sem_u).wait()
            pltpu.make_async_copy(
                it_t_hbm.at[:, pl.ds(0, TILE_W)], ring_i.at[0], sem_i).wait()

        def extract(j, slot):
            sv = jnp.full((LANES,), slot, jnp.int32)
            jv = jnp.full((LANES,), j, jnp.int32)
            cu = jnp.full((LANES,), ids_u[j] & (TILE_W - 1), jnp.int32)
            u0 = plsc.load_gather(ring_u, [sv, dlane, cu])
            u1 = plsc.load_gather(ring_u, [sv, dlane + LANES, cu])
            plsc.store_scatter(eu_v, [dlo + jv], u0)
            plsc.store_scatter(eu_v, [dhi + jv], u1)
            ci = jnp.full((LANES,), ids_i[j] & (TILE_W - 1), jnp.int32)
            i0 = plsc.load_gather(ring_i, [sv, dlane, ci])
            i1 = plsc.load_gather(ring_i, [sv, dlane + LANES, ci])
            plsc.store_scatter(ei_v, [dlo + jv], i0)
            plsc.store_scatter(ei_v, [dhi + jv], i1)

        for s in range(NSLOT - 1):
            fire(s, s)

        def gather_body(j, _):
            @pl.when(j + NSLOT - 1 < b_per_w)
            def _():
                fire(j + NSLOT - 1, lax.rem(j + NSLOT - 1, NSLOT))

            drain_one()
            extract(j, lax.rem(j, NSLOT))
            return 0

        lax.fori_loop(0, b_per_w, gather_body, 0)

        def blk_body(b, _):
            acc = jnp.zeros((LANES,), jnp.float32)
            for d in range(D):
                o = d * b_per_w + b * LANES
                acc = acc + (eu_v[pl.ds(o, LANES)] * ei_v[pl.ds(o, LANES)])
            rec_v[pl.ds(b * LANES, LANES)] = acc
            return 0

        lax.fori_loop(0, n_blk, blk_body, 0)

        out_copies = []
        for d in range(D):
            out_copies.append(pltpu.async_copy(
                eu_v.at[pl.ds(d * b_per_w, b_per_w)],
                eu_out.at[pl.ds(d * B + base, b_per_w)], sem_idx))
            out_copies.append(pltpu.async_copy(
                ei_v.at[pl.ds(d * b_per_w, b_per_w)],
                ei_out.at[pl.ds(d * B + base, b_per_w)], sem_idx))
        pltpu.sync_copy(rec_v, rec_out.at[pl.ds(base, b_per_w)])
        for cp in out_copies:
            cp.wait()

    return sc_kernel


def kernel(user, item, user_table, item_table):
    B = user.shape[0]
    N, D = user_table.shape
    sc = _make_sc_kernel(B, D, N)
    eu_f, ei_f, rec = sc(user.astype(jnp.int32), item.astype(jnp.int32),
                         user_table.T, item_table.T)
    return (eu_f.reshape(D, B).T, ei_f.reshape(D, B).T, rec)


# speedup vs baseline: 4.4537x; 1.0007x over previous
import functools

import jax
import jax.numpy as jnp
from jax import lax
from jax.experimental import pallas as pl
from jax.experimental.pallas import tpu as pltpu
from jax.experimental.pallas import tpu_sc as plsc

NUM_CORES = 2
NUM_SUBCORES = 16
LANES = 16
NSLOT = 11
TILE_W = 128


def _make_sc_kernel(B, D, N):
    NW = NUM_CORES * NUM_SUBCORES
    b_per_w = B // NW
    n_blk = b_per_w // LANES

    mesh = plsc.VectorSubcoreMesh(
        core_axis_name="c", subcore_axis_name="s",
        num_cores=NUM_CORES, num_subcores=NUM_SUBCORES)

    @functools.partial(
        pl.kernel,
        mesh=mesh,
        out_type=(
            jax.ShapeDtypeStruct((D * B,), jnp.float32),
            jax.ShapeDtypeStruct((D * B,), jnp.float32),
            jax.ShapeDtypeStruct((B,), jnp.float32),
        ),
        scratch_types=[
            pltpu.SMEM((b_per_w,), jnp.int32),
            pltpu.SMEM((b_per_w,), jnp.int32),
            pltpu.VMEM((b_per_w,), jnp.int32),
            pltpu.VMEM((b_per_w,), jnp.int32),
            pltpu.VMEM((NSLOT, D, TILE_W), jnp.float32),
            pltpu.VMEM((NSLOT, D, TILE_W), jnp.float32),
            pltpu.VMEM((D * b_per_w,), jnp.float32),
            pltpu.VMEM((D * b_per_w,), jnp.float32),
            pltpu.VMEM((b_per_w,), jnp.float32),
            pltpu.SemaphoreType.DMA,
            pltpu.SemaphoreType.DMA,
            pltpu.SemaphoreType.DMA,
        ],
        compiler_params=pltpu.CompilerParams(
            needs_layout_passes=False, use_tc_tiling_on_sc=True),
    )
    def sc_kernel(user_hbm, item_hbm, ut_t_hbm, it_t_hbm,
                  eu_out, ei_out, rec_out,
                  ids_u, ids_i, idsv_u, idsv_i, ring_u, ring_i,
                  eu_v, ei_v, rec_v,
                  sem_idx, sem_u, sem_i):
        wid = lax.axis_index("s") * NUM_CORES + lax.axis_index("c")
        base = wid * b_per_w

        cp_u = pltpu.async_copy(
            user_hbm.at[pl.ds(base, b_per_w)], idsv_u, sem_idx)
        cp_i = pltpu.async_copy(
            item_hbm.at[pl.ds(base, b_per_w)], idsv_i, sem_idx)
        cp_u.wait()
        cp_i.wait()
        def smem_body(grp, _):
            o = grp * LANES
            vu = idsv_u[pl.ds(o, LANES)]
            vi = idsv_i[pl.ds(o, LANES)]
            for k in range(LANES):
                ids_u[o + k] = vu[k]
                ids_i[o + k] = vi[k]
            return 0

        lax.fori_loop(0, b_per_w // LANES, smem_body, 0)

        dlane = lax.iota(jnp.int32, LANES)
        dlo = dlane * b_per_w
        dhi = (dlane + LANES) * b_per_w

        def fire(j, slot):
            tu = pl.multiple_of((ids_u[j] >> 7) * TILE_W, TILE_W)
            pltpu.async_copy(
                ut_t_hbm.at[:, pl.ds(tu, TILE_W)], ring_u.at[slot], sem_u)
            ti = pl.multiple_of((ids_i[j] >> 7) * TILE_W, TILE_W)
            pltpu.async_copy(
                it_t_hbm.at[:, pl.ds(ti, TILE_W)], ring_i.at[slot], sem_i)

        def drain_one():
            pltpu.make_async_copy(
                ut_t_hbm.at[:, pl.ds(0, TILE_W)], ring_u.at[0], sem_u).wait()
            pltpu.make_async_copy(
                it_t_hbm.at[:, pl.ds(0, TILE_W)], ring_i.at[0], sem_i).wait()

        def extract(j, slot):
            sv = jnp.full((LANES,), slot, jnp.int32)
            jv = jnp.full((LANES,), j, jnp.int32)
            cu = jnp.full((LANES,), ids_u[j] & (TILE_W - 1), jnp.int32)
            u0 = plsc.load_gather(ring_u, [sv, dlane, cu])
            u1 = plsc.load_gather(ring_u, [sv, dlane + LANES, cu])
            plsc.store_scatter(eu_v, [dlo + jv], u0)
            plsc.store_scatter(eu_v, [dhi + jv], u1)
            ci = jnp.full((LANES,), ids_i[j] & (TILE_W - 1), jnp.int32)
            i0 = plsc.load_gather(ring_i, [sv, dlane, ci])
            i1 = plsc.load_gather(ring_i, [sv, dlane + LANES, ci])
            plsc.store_scatter(ei_v, [dlo + jv], i0)
            plsc.store_scatter(ei_v, [dhi + jv], i1)

        for s in range(NSLOT - 1):
            fire(s, s)

        def gather_body(j, _):
            @pl.when(j + NSLOT - 1 < b_per_w)
            def _():
                fire(j + NSLOT - 1, lax.rem(j + NSLOT - 1, NSLOT))

            drain_one()
            extract(j, lax.rem(j, NSLOT))
            return 0

        lax.fori_loop(0, b_per_w, gather_body, 0)

        def blk_body(b, _):
            acc = jnp.zeros((LANES,), jnp.float32)
            for d in range(D):
                o = d * b_per_w + b * LANES
                acc = acc + (eu_v[pl.ds(o, LANES)] * ei_v[pl.ds(o, LANES)])
            rec_v[pl.ds(b * LANES, LANES)] = acc
            return 0

        lax.fori_loop(0, n_blk, blk_body, 0)

        out_copies = []
        for d in range(D):
            out_copies.append(pltpu.async_copy(
                eu_v.at[pl.ds(d * b_per_w, b_per_w)],
                eu_out.at[pl.ds(d * B + base, b_per_w)], sem_idx))
            out_copies.append(pltpu.async_copy(
                ei_v.at[pl.ds(d * b_per_w, b_per_w)],
                ei_out.at[pl.ds(d * B + base, b_per_w)], sem_idx))
        pltpu.sync_copy(rec_v, rec_out.at[pl.ds(base, b_per_w)])
        for cp in out_copies:
            cp.wait()

    return sc_kernel


def kernel(user, item, user_table, item_table):
    B = user.shape[0]
    N, D = user_table.shape
    sc = _make_sc_kernel(B, D, N)
    eu_f, ei_f, rec = sc(user.astype(jnp.int32), item.astype(jnp.int32),
                         user_table.T, item_table.T)
    return (eu_f.reshape(D, B).T, ei_f.reshape(D, B).T, rec)


# speedup vs baseline: 4.4589x; 1.0012x over previous
import functools

import jax
import jax.numpy as jnp
from jax import lax
from jax.experimental import pallas as pl
from jax.experimental.pallas import tpu as pltpu
from jax.experimental.pallas import tpu_sc as plsc

NUM_CORES = 2
NUM_SUBCORES = 16
LANES = 16
NSLOT = 10
TILE_W = 128


def _make_sc_kernel(B, D, N):
    NW = NUM_CORES * NUM_SUBCORES
    b_per_w = B // NW
    n_blk = b_per_w // LANES

    mesh = plsc.VectorSubcoreMesh(
        core_axis_name="c", subcore_axis_name="s",
        num_cores=NUM_CORES, num_subcores=NUM_SUBCORES)

    @functools.partial(
        pl.kernel,
        mesh=mesh,
        out_type=(
            jax.ShapeDtypeStruct((D * B,), jnp.float32),
            jax.ShapeDtypeStruct((D * B,), jnp.float32),
            jax.ShapeDtypeStruct((B,), jnp.float32),
        ),
        scratch_types=[
            pltpu.SMEM((b_per_w,), jnp.int32),
            pltpu.SMEM((b_per_w,), jnp.int32),
            pltpu.VMEM((b_per_w,), jnp.int32),
            pltpu.VMEM((b_per_w,), jnp.int32),
            pltpu.VMEM((NSLOT, D, TILE_W), jnp.float32),
            pltpu.VMEM((NSLOT, D, TILE_W), jnp.float32),
            pltpu.VMEM((D * b_per_w,), jnp.float32),
            pltpu.VMEM((D * b_per_w,), jnp.float32),
            pltpu.VMEM((b_per_w,), jnp.float32),
            pltpu.SemaphoreType.DMA,
            pltpu.SemaphoreType.DMA,
            pltpu.SemaphoreType.DMA,
        ],
        compiler_params=pltpu.CompilerParams(
            needs_layout_passes=False, use_tc_tiling_on_sc=True),
    )
    def sc_kernel(user_hbm, item_hbm, ut_t_hbm, it_t_hbm,
                  eu_out, ei_out, rec_out,
                  ids_u, ids_i, idsv_u, idsv_i, ring_u, ring_i,
                  eu_v, ei_v, rec_v,
                  sem_idx, sem_u, sem_i):
        wid = lax.axis_index("s") * NUM_CORES + lax.axis_index("c")
        base = wid * b_per_w

        cp_u = pltpu.async_copy(
            user_hbm.at[pl.ds(base, b_per_w)], idsv_u, sem_idx)
        cp_i = pltpu.async_copy(
            item_hbm.at[pl.ds(base, b_per_w)], idsv_i, sem_idx)
        cp_u.wait()
        cp_i.wait()
        def smem_body(grp, _):
            o = grp * LANES
            vu = idsv_u[pl.ds(o, LANES)]
            vi = idsv_i[pl.ds(o, LANES)]
            for k in range(LANES):
                ids_u[o + k] = vu[k]
                ids_i[o + k] = vi[k]
            return 0

        lax.fori_loop(0, b_per_w // LANES, smem_body, 0)

        dlane = lax.iota(jnp.int32, LANES)
        dlo = dlane * b_per_w
        dhi = (dlane + LANES) * b_per_w

        def fire(j, slot):
            tu = pl.multiple_of((ids_u[j] >> 7) * TILE_W, TILE_W)
            pltpu.async_copy(
                ut_t_hbm.at[:, pl.ds(tu, TILE_W)], ring_u.at[slot], sem_u)
            ti = pl.multiple_of((ids_i[j] >> 7) * TILE_W, TILE_W)
            pltpu.async_copy(
                it_t_hbm.at[:, pl.ds(ti, TILE_W)], ring_i.at[slot], sem_i)

        def drain_one():
            pltpu.make_async_copy(
                ut_t_hbm.at[:, pl.ds(0, TILE_W)], ring_u.at[0], sem_u).wait()
            pltpu.make_async_copy(
                it_t_hbm.at[:, pl.ds(0, TILE_W)], ring_i.at[0], sem_i).wait()

        def extract(j, slot):
            sv = jnp.full((LANES,), slot, jnp.int32)
            jv = jnp.full((LANES,), j, jnp.int32)
            cu = jnp.full((LANES,), ids_u[j] & (TILE_W - 1), jnp.int32)
            u0 = plsc.load_gather(ring_u, [sv, dlane, cu])
            u1 = plsc.load_gather(ring_u, [sv, dlane + LANES, cu])
            plsc.store_scatter(eu_v, [dlo + jv], u0)
            plsc.store_scatter(eu_v, [dhi + jv], u1)
            ci = jnp.full((LANES,), ids_i[j] & (TILE_W - 1), jnp.int32)
            i0 = plsc.load_gather(ring_i, [sv, dlane, ci])
            i1 = plsc.load_gather(ring_i, [sv, dlane + LANES, ci])
            plsc.store_scatter(ei_v, [dlo + jv], i0)
            plsc.store_scatter(ei_v, [dhi + jv], i1)

        for s in range(NSLOT - 1):
            fire(s, s)

        def gather_body(j, _):
            @pl.when(j + NSLOT - 1 < b_per_w)
            def _():
                fire(j + NSLOT - 1, lax.rem(j + NSLOT - 1, NSLOT))

            drain_one()
            extract(j, lax.rem(j, NSLOT))
            return 0

        lax.fori_loop(0, b_per_w, gather_body, 0)

        def blk_body(b, _):
            acc = jnp.zeros((LANES,), jnp.float32)
            for d in range(D):
                o = d * b_per_w + b * LANES
                acc = acc + (eu_v[pl.ds(o, LANES)] * ei_v[pl.ds(o, LANES)])
            rec_v[pl.ds(b * LANES, LANES)] = acc
            return 0

        lax.fori_loop(0, n_blk, blk_body, 0)

        out_copies = []
        for d in range(D):
            out_copies.append(pltpu.async_copy(
                eu_v.at[pl.ds(d * b_per_w, b_per_w)],
                eu_out.at[pl.ds(d * B + base, b_per_w)], sem_idx))
            out_copies.append(pltpu.async_copy(
                ei_v.at[pl.ds(d * b_per_w, b_per_w)],
                ei_out.at[pl.ds(d * B + base, b_per_w)], sem_idx))
        pltpu.sync_copy(rec_v, rec_out.at[pl.ds(base, b_per_w)])
        for cp in out_copies:
            cp.wait()

    return sc_kernel


def kernel(user, item, user_table, item_table):
    B = user.shape[0]
    N, D = user_table.shape
    sc = _make_sc_kernel(B, D, N)
    eu_f, ei_f, rec = sc(user.astype(jnp.int32), item.astype(jnp.int32),
                         user_table.T, item_table.T)
    return (eu_f.reshape(D, B).T, ei_f.reshape(D, B).T, rec)
